# baseline (speedup 1.0000x reference)
import functools

import jax
import jax.numpy as jnp
from jax import lax
from jax.experimental import pallas as pl
from jax.experimental.pallas import tpu as pltpu
from jax.experimental.pallas import tpu_sc as plsc

_B = 16384
_LSEQ = 200
_V = 1000
_VPAD = 1024
_D = 16
_NOUT = 3

_NC = 2
_NS = 16
_NW = _NC * _NS
_RPW = _B // _NW
_CSEQ = 40
_NCHUNK = _LSEQ // _CSEQ
_NGROUP = _RPW // 16
_VCHUNK = (_V + 15) // 16
_UNROLL = 8

_sc_mesh = plsc.VectorSubcoreMesh(core_axis_name="c", subcore_axis_name="s")


@functools.partial(
    pl.kernel,
    mesh=_sc_mesh,
    out_type=jax.ShapeDtypeStruct((_NOUT, _B), jnp.float32),
    scratch_types=[
        pltpu.VMEM((2 * _VPAD,), jnp.float32),
        pltpu.VMEM((_D, _V), jnp.float32),
        pltpu.VMEM((_NOUT, _D), jnp.float32),
        pltpu.VMEM((_CSEQ, _RPW), jnp.int32),
        pltpu.VMEM((_CSEQ, _RPW), jnp.int32),
        pltpu.VMEM((_NOUT, _RPW), jnp.float32),
        pltpu.SemaphoreType.DMA,
        pltpu.SemaphoreType.DMA,
    ],
    compiler_params=pltpu.CompilerParams(needs_layout_passes=False),
)
def _sc_pool(xt_hbm, embt_hbm, w_hbm, out_hbm,
             tab_v, embt_v, w_v, x_v0, x_v1, out_v, sem0, sem1):
    wid = lax.axis_index("s") * _NC + lax.axis_index("c")
    row0 = wid * _RPW

    bufs = (x_v0, x_v1)
    sems = (sem0, sem1)
    copies = [None, None]

    def start_chunk(c):
        b = c % 2
        copies[b] = pltpu.async_copy(
            xt_hbm.at[pl.ds(c * _CSEQ, _CSEQ), pl.ds(row0, _RPW)],
            bufs[b], sems[b])

    start_chunk(0)
    pltpu.sync_copy(embt_hbm, embt_v)
    pltpu.sync_copy(w_hbm, w_v)

    scale = 1.0 / _LSEQ
    rne = jnp.full((16,), 0x7FFF, dtype=jnp.int32)
    one = jnp.full((16,), 1, dtype=jnp.int32)
    himask = jnp.full((16,), -65536, dtype=jnp.int32)
    off2 = jnp.full((16,), _VPAD, dtype=jnp.int32)
    fzero = jnp.zeros((16,), jnp.float32)

    def build_body(v0, carry):
        base = jnp.minimum(v0 * 16, _V - 16)
        wrows = [w_v[j, pl.ds(0, _D)] * scale for j in range(_NOUT)]
        accs = [fzero, fzero, fzero]
        for d in range(_D):
            e = embt_v[d, pl.ds(base, 16)]
            for j in range(_NOUT):
                accs[j] = accs[j] + e * wrows[j][d]

        def to_bf16_bits(a):
            u = plsc.bitcast(a, jnp.int32)
            return u + rne + (lax.shift_right_logical(u, 16) & one)

        p = (lax.shift_right_logical(to_bf16_bits(accs[0]), 16)
             | (to_bf16_bits(accs[1]) & himask))
        tab_v[pl.ds(base, 16)] = plsc.bitcast(p, jnp.float32)
        tab_v[pl.ds(_VPAD + base, 16)] = accs[2]
        return carry

    lax.fori_loop(0, _VCHUNK, build_body, 0)

    for c in range(_NCHUNK):
        copies[c % 2].wait()
        if c + 1 < _NCHUNK:
            start_chunk(c + 1)
        x_v = bufs[c % 2]

        def group_body(g, carry, x_v=x_v, c=c):
            gbase = g * 16

            @plsc.parallel_loop(0, _CSEQ, unroll=_UNROLL,
                                carry=(fzero, fzero, fzero))
            def l_body(l, accs):
                a0, a1, a2 = accs
                xi = x_v[l, pl.ds(gbase, 16)]
                gp = plsc.bitcast(plsc.load_gather(tab_v, [xi]), jnp.int32)
                a0 = a0 + plsc.bitcast(gp << 16, jnp.float32)
                a1 = a1 + plsc.bitcast(gp & himask, jnp.float32)
                a2 = a2 + plsc.load_gather(tab_v, [xi + off2])
                return (a0, a1, a2)

            a0, a1, a2 = l_body
            if c == 0:
                out_v[0, pl.ds(gbase, 16)] = a0
                out_v[1, pl.ds(gbase, 16)] = a1
                out_v[2, pl.ds(gbase, 16)] = a2
            else:
                out_v[0, pl.ds(gbase, 16)] = out_v[0, pl.ds(gbase, 16)] + a0
                out_v[1, pl.ds(gbase, 16)] = out_v[1, pl.ds(gbase, 16)] + a1
                out_v[2, pl.ds(gbase, 16)] = out_v[2, pl.ds(gbase, 16)] + a2
            return carry

        lax.fori_loop(0, _NGROUP, group_body, 0)

    pltpu.sync_copy(out_v, out_hbm.at[:, pl.ds(row0, _RPW)])


def kernel(x, emb_table, fc_w, fc_b):
    out3 = _sc_pool(x.T, emb_table.T, fc_w)
    return out3.T + fc_b


# speedup vs baseline: 1.0694x; 1.0694x over previous
import functools

import jax
import jax.numpy as jnp
from jax import lax
from jax.experimental import pallas as pl
from jax.experimental.pallas import tpu as pltpu
from jax.experimental.pallas import tpu_sc as plsc

_B = 16384
_LSEQ = 200
_V = 1000
_D = 16
_NOUT = 3

_NC = 2
_NS = 16
_NW = _NC * _NS
_RPW = _B // _NW
_CSEQ = 40
_NCHUNK = _LSEQ // _CSEQ
_NGROUP = _RPW // 16
_UNROLL = 8


def _tab_kernel(emb_ref, w_ref, b_ref, out_ref):
    t = lax.dot_general(
        w_ref[...], emb_ref[...], (((1,), (1,)), ((), ())),
        preferred_element_type=jnp.float32,
    ) * (1.0 / _LSEQ) + b_ref[...] * (1.0 / _LSEQ)
    b0 = lax.bitcast_convert_type(
        t[0:1, :].astype(jnp.bfloat16), jnp.uint16).astype(jnp.uint32)
    b1 = lax.bitcast_convert_type(
        t[1:2, :].astype(jnp.bfloat16), jnp.uint16).astype(jnp.uint32)
    packed = lax.bitcast_convert_type(b0 | (b1 << 16), jnp.float32)
    out_ref[0:1, :] = packed
    out_ref[1:2, :] = t[2:3, :]


_sc_mesh = plsc.VectorSubcoreMesh(core_axis_name="c", subcore_axis_name="s")


@functools.partial(
    pl.kernel,
    mesh=_sc_mesh,
    out_type=jax.ShapeDtypeStruct((_NOUT, _B), jnp.float32),
    scratch_types=[
        pltpu.VMEM((2 * _V,), jnp.float32),
        pltpu.VMEM((_CSEQ, _RPW), jnp.int32),
        pltpu.VMEM((_CSEQ, _RPW), jnp.int32),
        pltpu.VMEM((_NOUT, _RPW), jnp.float32),
        pltpu.SemaphoreType.DMA,
        pltpu.SemaphoreType.DMA,
    ],
    compiler_params=pltpu.CompilerParams(needs_layout_passes=False),
)
def _sc_pool(xt_hbm, tab_hbm, out_hbm, tab_v, x_v0, x_v1, out_v, sem0, sem1):
    wid = lax.axis_index("s") * _NC + lax.axis_index("c")
    row0 = wid * _RPW
    pltpu.sync_copy(tab_hbm, tab_v)

    bufs = (x_v0, x_v1)
    sems = (sem0, sem1)
    copies = [None, None]

    def start_chunk(c):
        b = c % 2
        copies[b] = pltpu.async_copy(
            xt_hbm.at[pl.ds(c * _CSEQ, _CSEQ), pl.ds(row0, _RPW)],
            bufs[b], sems[b])

    start_chunk(0)

    off2 = jnp.full((16,), _V, dtype=jnp.int32)
    himask = jnp.full((16,), -65536, dtype=jnp.int32)
    fzero = jnp.zeros((16,), jnp.float32)

    for c in range(_NCHUNK):
        copies[c % 2].wait()
        if c + 1 < _NCHUNK:
            start_chunk(c + 1)
        x_v = bufs[c % 2]

        def group_body(g, carry, x_v=x_v, c=c):
            gbase = g * 16

            @plsc.parallel_loop(0, _CSEQ, unroll=_UNROLL,
                                carry=(fzero, fzero, fzero))
            def l_body(l, accs):
                a0, a1, a2 = accs
                xi = x_v[l, pl.ds(gbase, 16)]
                gp = plsc.bitcast(plsc.load_gather(tab_v, [xi]), jnp.int32)
                a0 = a0 + plsc.bitcast(gp << 16, jnp.float32)
                a1 = a1 + plsc.bitcast(gp & himask, jnp.float32)
                a2 = a2 + plsc.load_gather(tab_v, [xi + off2])
                return (a0, a1, a2)

            a0, a1, a2 = l_body
            if c == 0:
                out_v[0, pl.ds(gbase, 16)] = a0
                out_v[1, pl.ds(gbase, 16)] = a1
                out_v[2, pl.ds(gbase, 16)] = a2
            else:
                out_v[0, pl.ds(gbase, 16)] = out_v[0, pl.ds(gbase, 16)] + a0
                out_v[1, pl.ds(gbase, 16)] = out_v[1, pl.ds(gbase, 16)] + a1
                out_v[2, pl.ds(gbase, 16)] = out_v[2, pl.ds(gbase, 16)] + a2
            return carry

        lax.fori_loop(0, _NGROUP, group_body, 0)

    pltpu.sync_copy(out_v, out_hbm.at[:, pl.ds(row0, _RPW)])


def kernel(x, emb_table, fc_w, fc_b):
    tab = pl.pallas_call(
        _tab_kernel,
        out_shape=jax.ShapeDtypeStruct((2, _V), jnp.float32),
    )(emb_table, fc_w, fc_b[:, None])
    out3 = _sc_pool(x.T, tab.reshape(2 * _V))
    return out3.T
